# Initial kernel scaffold; baseline (speedup 1.0000x reference)
import jax
import jax.numpy as jnp
from jax import lax
from jax.experimental import pallas as pl
from jax.experimental.pallas import tpu as pltpu
from jax.experimental.pallas import tpu_sc as plsc

P = 10000
M = 32
D = 128

NC = 2
NS = 16
NW = NC * NS
PW = 320
P_PAD = NW * PW
CP = 4
NCH = PW // CP
DG = D // 16


def _sc_body(table, adj, w, out_hbm,
             idxv, wv, rv, rows0, rows1, feat0, feat1, ob0, ob1,
             gs0, gs1, fs0, fs1, os0, os1):
    c = lax.axis_index("c")
    s = lax.axis_index("s")
    wid = s * NC + c
    base = wid * PW

    pltpu.sync_copy(adj.at[wid], idxv)
    pltpu.sync_copy(w.at[wid], wv)

    @pl.loop(0, PW // 16)
    def _(g):
        w16 = wv[pl.ds(g * 16, 16)]
        rv[pl.ds(g * 16, 16)] = 1.0 / (w16 + 1e-10)

    rows = [rows0, rows1]
    feat = [feat0, feat1]
    ob = [ob0, ob1]
    gs = [gs0, gs1]
    fs = [fs0, fs1]
    osm = [os0, os1]

    def issue(g, b):
        pltpu.async_copy(table.at[idxv.at[g]], rows[b], gs[b])
        pltpu.async_copy(table.at[pl.ds(base + g * CP + 1, CP)], feat[b], fs[b])

    issue(0, 0)
    issue(1, 1)

    @pl.loop(0, NCH, step=2)
    def _(g0):
        for b in range(2):
            g = g0 + b
            pltpu.make_async_copy(table.at[idxv.at[g]], rows[b], gs[b]).wait()
            pltpu.make_async_copy(
                table.at[pl.ds(base + g * CP + 1, CP)], feat[b], fs[b]).wait()

            @pl.when(g >= 2)
            def _():
                pltpu.make_async_copy(
                    ob[b], out_hbm.at[pl.ds(base, CP)], osm[b]).wait()

            @pl.loop(0, CP)
            def _(i):
                r0 = i * M
                acc = [rows[b][r0, pl.ds(d * 16, 16)] for d in range(DG)]
                for m in range(1, M):
                    for d in range(DG):
                        acc[d] = acc[d] + rows[b][r0 + m, pl.ds(d * 16, 16)]
                ivec = jnp.full((16,), 0, jnp.int32) + (g * CP + i)
                rb = plsc.load_gather(rv, [ivec])
                for d in range(DG):
                    nei = acc[d] * rb
                    df = nei - feat[b][i, pl.ds(d * 16, 16)]
                    ob[b][i, pl.ds(d * 16, 16)] = df * df

            pltpu.async_copy(ob[b], out_hbm.at[pl.ds(base + g * CP, CP)], osm[b])

            @pl.when(g + 2 < NCH)
            def _():
                issue(g + 2, b)

    for b in range(2):
        pltpu.make_async_copy(ob[b], out_hbm.at[pl.ds(base, CP)], osm[b]).wait()


_sc_kernel = pl.kernel(
    _sc_body,
    out_type=jax.ShapeDtypeStruct((P_PAD, D), jnp.float32),
    mesh=plsc.VectorSubcoreMesh(
        core_axis_name="c", subcore_axis_name="s", num_cores=NC, num_subcores=NS),
    scratch_types=[
        pltpu.VMEM((NCH, CP * M), jnp.int32),
        pltpu.VMEM((PW,), jnp.float32),
        pltpu.VMEM((PW,), jnp.float32),
        pltpu.VMEM((CP * M, D), jnp.float32),
        pltpu.VMEM((CP * M, D), jnp.float32),
        pltpu.VMEM((CP, D), jnp.float32),
        pltpu.VMEM((CP, D), jnp.float32),
        pltpu.VMEM((CP, D), jnp.float32),
        pltpu.VMEM((CP, D), jnp.float32),
        pltpu.SemaphoreType.DMA,
        pltpu.SemaphoreType.DMA,
        pltpu.SemaphoreType.DMA,
        pltpu.SemaphoreType.DMA,
        pltpu.SemaphoreType.DMA,
        pltpu.SemaphoreType.DMA,
    ],
)


@jax.jit
def kernel(pointfeat, adj_idx, adj_weights):
    table = jnp.zeros((P_PAD + 1, D), jnp.float32).at[1:P + 1].set(pointfeat)
    adj_p = jnp.zeros((P_PAD, M), jnp.int32).at[:P].set(adj_idx)
    adj_r = adj_p.reshape(NW, NCH, CP * M)
    w_p = jnp.zeros((P_PAD,), jnp.float32).at[:P].set(adj_weights[:, 0])
    w_r = w_p.reshape(NW, PW)
    out = _sc_kernel(table, adj_r, w_r)
    return out[:P]


# speedup vs baseline: 1.5480x; 1.5480x over previous
import jax
import jax.numpy as jnp
from jax import lax
from jax.experimental import pallas as pl
from jax.experimental.pallas import tpu as pltpu
from jax.experimental.pallas import tpu_sc as plsc

P = 10000
M = 32
D = 128

NC = 2
NS = 16
NW = NC * NS
PW = 320
P_PAD = NW * PW
CP = 8
NCH = PW // CP
HALF = CP // 2 * M
DG = D // 16


def _sc_body(table, adj, featg, w, out_hbm,
             idxv, wv, ra0, rb0, ra1, rb1, f0, f1, ob0, ob1,
             ga0, gb0, ga1, gb1, fs0, fs1, os0, os1):
    c = lax.axis_index("c")
    s = lax.axis_index("s")
    wid = s * NC + c
    base = wid * PW

    pltpu.sync_copy(adj.at[wid], idxv)
    pltpu.sync_copy(w.at[wid], wv)

    ra = [ra0, ra1]
    rb = [rb0, rb1]
    ft = [f0, f1]
    ob = [ob0, ob1]
    ga = [ga0, ga1]
    gb = [gb0, gb1]
    fs = [fs0, fs1]
    osm = [os0, os1]

    def issue(g, b):
        pltpu.async_copy(table.at[idxv.at[2 * g]], ra[b], ga[b])
        pltpu.async_copy(table.at[idxv.at[2 * g + 1]], rb[b], gb[b])
        pltpu.async_copy(featg.at[pl.ds(base + g * CP, CP)], ft[b], fs[b])

    issue(0, 0)
    issue(1, 1)

    @pl.loop(0, NCH, step=2)
    def _(g0):
        for b in range(2):
            g = g0 + b
            pltpu.make_async_copy(table.at[idxv.at[2 * g]], ra[b], ga[b]).wait()
            pltpu.make_async_copy(table.at[idxv.at[2 * g]], rb[b], gb[b]).wait()
            pltpu.make_async_copy(
                featg.at[pl.ds(base + g * CP, CP)], ft[b], fs[b]).wait()

            @pl.when(g >= 2)
            def _():
                pltpu.make_async_copy(
                    ob[b], out_hbm.at[pl.ds(base, CP)], osm[b]).wait()

            for half, rbuf in ((0, ra[b]), (1, rb[b])):
                @pl.loop(0, CP // 2)
                def _(i):
                    r0 = i * M
                    acc = [rbuf[r0, pl.ds(d * 16, 16)] for d in range(DG)]
                    for m in range(1, M):
                        for d in range(DG):
                            acc[d] = acc[d] + rbuf[r0 + m, pl.ds(d * 16, 16)]
                    pidx = g * CP + half * (CP // 2) + i
                    rw = 1.0 / (wv[pidx, pl.ds(0, 16)] + 1e-10)
                    orow = half * (CP // 2) + i
                    for d in range(DG):
                        nei = acc[d] * rw
                        df = nei - ft[b][orow, pl.ds(d * 16, 16)]
                        ob[b][orow, pl.ds(d * 16, 16)] = df * df

            pltpu.async_copy(ob[b], out_hbm.at[pl.ds(base + g * CP, CP)], osm[b])

            @pl.when(g + 2 < NCH)
            def _():
                issue(g + 2, b)

    for b in range(2):
        pltpu.make_async_copy(ob[b], out_hbm.at[pl.ds(base, CP)], osm[b]).wait()


_sc_kernel = pl.kernel(
    _sc_body,
    out_type=jax.ShapeDtypeStruct((P_PAD, D), jnp.float32),
    mesh=plsc.VectorSubcoreMesh(
        core_axis_name="c", subcore_axis_name="s", num_cores=NC, num_subcores=NS),
    scratch_types=[
        pltpu.VMEM((2 * NCH, HALF), jnp.int32),
        pltpu.VMEM((PW, 16), jnp.float32),
        pltpu.VMEM((HALF, D), jnp.float32),
        pltpu.VMEM((HALF, D), jnp.float32),
        pltpu.VMEM((HALF, D), jnp.float32),
        pltpu.VMEM((HALF, D), jnp.float32),
        pltpu.VMEM((CP, D), jnp.float32),
        pltpu.VMEM((CP, D), jnp.float32),
        pltpu.VMEM((CP, D), jnp.float32),
        pltpu.VMEM((CP, D), jnp.float32),
        pltpu.SemaphoreType.DMA,
        pltpu.SemaphoreType.DMA,
        pltpu.SemaphoreType.DMA,
        pltpu.SemaphoreType.DMA,
        pltpu.SemaphoreType.DMA,
        pltpu.SemaphoreType.DMA,
        pltpu.SemaphoreType.DMA,
        pltpu.SemaphoreType.DMA,
    ],
)


@jax.jit
def kernel(pointfeat, adj_idx, adj_weights):
    table = jnp.zeros((P + 1, D), jnp.float32).at[1:].set(pointfeat)
    adj_p = jnp.zeros((P_PAD, M), jnp.int32).at[:P].set(adj_idx)
    adj_r = adj_p.reshape(NW, 2 * NCH, HALF)
    feat_p = jnp.zeros((P_PAD, D), jnp.float32).at[:P].set(pointfeat)
    w_p = jnp.zeros((P_PAD,), jnp.float32).at[:P].set(adj_weights[:, 0])
    w_r = jnp.broadcast_to(w_p[:, None], (P_PAD, 16)).reshape(NW, PW, 16)
    out = _sc_kernel(table, adj_r, feat_p, w_r)
    return out[:P]


# speedup vs baseline: 5.1506x; 3.3273x over previous
import jax
import jax.numpy as jnp
from jax import lax
from jax.experimental import pallas as pl
from jax.experimental.pallas import tpu as pltpu
from jax.experimental.pallas import tpu_sc as plsc

P = 10000
M = 32
D = 128

NC = 2
NS = 16
NW = NC * NS
PW = 320
P_PAD = NW * PW
CP = 4
NCH = PW // CP
OG = 8
NGP = PW // OG
DG = D // 16


def _sc_body(table, adj, featg, w, out_hbm,
             tsh, ix0, ix1, r0, r1, f0, f1, w0, w1, ob0, ob1,
             is0, is1, gs0, gs1, fs0, fs1, ws0, ws1, os0, os1):
    c = lax.axis_index("c")
    s = lax.axis_index("s")
    wid = s * NC + c
    base = wid * PW

    @pl.when(s == 0)
    def _():
        pltpu.sync_copy(table, tsh)
    plsc.subcore_barrier()

    ix = [ix0, ix1]
    rows = [r0, r1]
    ft = [f0, f1]
    wb = [w0, w1]
    ob = [ob0, ob1]
    isx = [is0, is1]
    gsm = [gs0, gs1]
    fsm = [fs0, fs1]
    wsm = [ws0, ws1]
    osm = [os0, os1]

    def issue_idx(g, b):
        pltpu.async_copy(adj.at[wid * NCH + g], ix[b], isx[b])

    def issue_gather(g, b):
        pltpu.async_copy(tsh.at[ix[b].at[0]], rows[b], gsm[b])

    def issue_group(gp, fb):
        pltpu.async_copy(featg.at[pl.ds(base + gp * OG, OG)], ft[fb], fsm[fb])
        pltpu.async_copy(w.at[wid * NGP + gp], wb[fb], wsm[fb])

    def wait_idx(g, b):
        pltpu.make_async_copy(adj.at[wid * NCH + g], ix[b], isx[b]).wait()

    def wait_gather(g, b):
        pltpu.make_async_copy(tsh.at[ix[b].at[0]], rows[b], gsm[b]).wait()

    def wait_group(gp, fb):
        pltpu.make_async_copy(
            featg.at[pl.ds(base + gp * OG, OG)], ft[fb], fsm[fb]).wait()
        pltpu.make_async_copy(w.at[wid * NGP + gp], wb[fb], wsm[fb]).wait()

    def wait_out(fb):
        pltpu.make_async_copy(ob[fb], out_hbm.at[pl.ds(base, OG)], osm[fb]).wait()

    def compute(g, b, fb, half):
        @pl.loop(0, CP)
        def _(i):
            ri = i * M
            rbuf = rows[b]
            acc = [rbuf[ri, pl.ds(d * 16, 16)] for d in range(DG)]
            for m in range(1, M):
                for d in range(DG):
                    acc[d] = acc[d] + rbuf[ri + m, pl.ds(d * 16, 16)]
            pg = half * CP + i
            rw = 1.0 / (wb[fb][pg, pl.ds(0, 16)] + 1e-10)
            for d in range(DG):
                nei = acc[d] * rw
                df = nei - ft[fb][pg, pl.ds(d * 16, 16)]
                ob[fb][pg, pl.ds(d * 16, 16)] = df * df

    issue_idx(0, 0)
    wait_idx(0, 0)
    issue_idx(1, 1)
    issue_gather(0, 0)
    issue_group(0, 0)
    issue_group(1, 1)

    @pl.loop(0, NCH, step=4)
    def _(g0):
        for sub in range(2):
            gp = g0 // 2 + sub
            fb0 = sub
            g = g0 + 2 * sub
            wait_idx(g + 1, 1)
            issue_gather(g + 1, 1)
            wait_gather(g, 0)

            @pl.when(g + 2 < NCH)
            def _():
                issue_idx(g + 2, 0)

            wait_group(gp, fb0)

            @pl.when(gp >= 2)
            def _():
                wait_out(fb0)

            compute(g, 0, fb0, 0)

            g = g0 + 2 * sub + 1

            @pl.when(g + 1 < NCH)
            def _():
                wait_idx(g + 1, 0)
                issue_gather(g + 1, 0)

            wait_gather(g, 1)

            @pl.when(g + 2 < NCH)
            def _():
                issue_idx(g + 2, 1)

            compute(g, 1, fb0, 1)

            pltpu.async_copy(
                ob[fb0], out_hbm.at[pl.ds(base + gp * OG, OG)], osm[fb0])

            @pl.when(gp + 2 < NGP)
            def _():
                issue_group(gp + 2, fb0)

    wait_out(0)
    wait_out(1)


_sc_kernel = pl.kernel(
    _sc_body,
    out_type=jax.ShapeDtypeStruct((P_PAD, D), jnp.float32),
    mesh=plsc.VectorSubcoreMesh(
        core_axis_name="c", subcore_axis_name="s", num_cores=NC, num_subcores=NS),
    scratch_types=[
        pltpu.VMEM_SHARED((P + 1, D), jnp.float32),
        pltpu.VMEM((1, CP * M), jnp.int32),
        pltpu.VMEM((1, CP * M), jnp.int32),
        pltpu.VMEM((CP * M, D), jnp.float32),
        pltpu.VMEM((CP * M, D), jnp.float32),
        pltpu.VMEM((OG, D), jnp.float32),
        pltpu.VMEM((OG, D), jnp.float32),
        pltpu.VMEM((OG, 16), jnp.float32),
        pltpu.VMEM((OG, 16), jnp.float32),
        pltpu.VMEM((OG, D), jnp.float32),
        pltpu.VMEM((OG, D), jnp.float32),
        pltpu.SemaphoreType.DMA,
        pltpu.SemaphoreType.DMA,
        pltpu.SemaphoreType.DMA,
        pltpu.SemaphoreType.DMA,
        pltpu.SemaphoreType.DMA,
        pltpu.SemaphoreType.DMA,
        pltpu.SemaphoreType.DMA,
        pltpu.SemaphoreType.DMA,
        pltpu.SemaphoreType.DMA,
        pltpu.SemaphoreType.DMA,
    ],
)


@jax.jit
def kernel(pointfeat, adj_idx, adj_weights):
    table = jnp.zeros((P + 1, D), jnp.float32).at[1:].set(pointfeat)
    adj_p = jnp.zeros((P_PAD, M), jnp.int32).at[:P].set(adj_idx)
    adj_r = adj_p.reshape(NW * NCH, 1, CP * M)
    feat_p = jnp.zeros((P_PAD, D), jnp.float32).at[:P].set(pointfeat)
    w_p = jnp.zeros((P_PAD,), jnp.float32).at[:P].set(adj_weights[:, 0])
    w_r = jnp.broadcast_to(w_p[:, None], (P_PAD, 16)).reshape(NW * NGP, OG, 16)
    out = _sc_kernel(table, adj_r, feat_p, w_r)
    return out[:P]


# speedup vs baseline: 5.3728x; 1.0431x over previous
import jax
import jax.numpy as jnp
from jax import lax
from jax.experimental import pallas as pl
from jax.experimental.pallas import tpu as pltpu
from jax.experimental.pallas import tpu_sc as plsc

P = 10000
M = 32
D = 128

NC = 2
NS = 16
NW = NC * NS
PW = 320
P_PAD = NW * PW
CP = 4
NCH = PW // CP
OG = 8
NGP = PW // OG
DG = D // 16
TROWS = 10112
SROWS = TROWS // NS


def _sc_body(table, adj, w, out_hbm,
             tsh, ix0, ix1, r0, r1, f0, f1, w0, w1, ob0, ob1,
             is0, is1, gs0, gs1, fs0, fs1, ws0, ws1, os0, os1):
    c = lax.axis_index("c")
    s = lax.axis_index("s")
    wid = s * NC + c
    base = wid * PW

    pltpu.sync_copy(table.at[pl.ds(s * SROWS, SROWS)],
                    tsh.at[pl.ds(s * SROWS, SROWS)])
    plsc.subcore_barrier()

    ix = [ix0, ix1]
    rows = [r0, r1]
    ft = [f0, f1]
    wb = [w0, w1]
    ob = [ob0, ob1]
    isx = [is0, is1]
    gsm = [gs0, gs1]
    fsm = [fs0, fs1]
    wsm = [ws0, ws1]
    osm = [os0, os1]

    def issue_idx(g, b):
        pltpu.async_copy(adj.at[wid * NCH + g], ix[b], isx[b])

    def issue_gather(g, b):
        pltpu.async_copy(tsh.at[ix[b].at[0]], rows[b], gsm[b])

    def feat_idx(gp):
        lanes = lax.iota(jnp.int32, 16)
        return jnp.where(lanes < OG,
                         jnp.minimum(base + gp * OG + 1 + lanes, P), 0)

    def issue_group(gp, fb):
        pltpu.async_copy(tsh.at[feat_idx(gp)], ft[fb], fsm[fb])
        pltpu.async_copy(w.at[wid * NGP + gp], wb[fb], wsm[fb])

    def wait_idx(g, b):
        pltpu.make_async_copy(adj.at[wid * NCH + g], ix[b], isx[b]).wait()

    def wait_gather(g, b):
        pltpu.make_async_copy(tsh.at[ix[b].at[0]], rows[b], gsm[b]).wait()

    def wait_group(gp, fb):
        pltpu.make_async_copy(tsh.at[feat_idx(gp)], ft[fb], fsm[fb]).wait()
        pltpu.make_async_copy(w.at[wid * NGP + gp], wb[fb], wsm[fb]).wait()

    def wait_out(fb):
        pltpu.make_async_copy(ob[fb], out_hbm.at[pl.ds(base, OG)], osm[fb]).wait()

    def compute(g, b, fb, half):
        @pl.loop(0, CP)
        def _(i):
            ri = i * M
            rbuf = rows[b]
            acc = [rbuf[ri, pl.ds(d * 16, 16)] for d in range(DG)]
            for m in range(1, M):
                for d in range(DG):
                    acc[d] = acc[d] + rbuf[ri + m, pl.ds(d * 16, 16)]
            pg = half * CP + i
            rw = 1.0 / (wb[fb][pg, pl.ds(0, 16)] + 1e-10)
            for d in range(DG):
                nei = acc[d] * rw
                df = nei - ft[fb][pg, pl.ds(d * 16, 16)]
                ob[fb][pg, pl.ds(d * 16, 16)] = df * df

    issue_idx(0, 0)
    wait_idx(0, 0)
    issue_idx(1, 1)
    issue_gather(0, 0)
    issue_group(0, 0)
    issue_group(1, 1)

    @pl.loop(0, NCH, step=4)
    def _(g0):
        for sub in range(2):
            gp = g0 // 2 + sub
            fb0 = sub
            g = g0 + 2 * sub
            wait_idx(g + 1, 1)
            issue_gather(g + 1, 1)
            wait_gather(g, 0)

            @pl.when(g + 2 < NCH)
            def _():
                issue_idx(g + 2, 0)

            wait_group(gp, fb0)

            @pl.when(jnp.logical_and(gp >= 2, base + (gp - 2) * OG < P))
            def _():
                wait_out(fb0)

            compute(g, 0, fb0, 0)

            g = g0 + 2 * sub + 1

            @pl.when(g + 1 < NCH)
            def _():
                wait_idx(g + 1, 0)
                issue_gather(g + 1, 0)

            wait_gather(g, 1)

            @pl.when(g + 2 < NCH)
            def _():
                issue_idx(g + 2, 1)

            compute(g, 1, fb0, 1)

            @pl.when(base + gp * OG < P)
            def _():
                pltpu.async_copy(
                    ob[fb0], out_hbm.at[pl.ds(base + gp * OG, OG)], osm[fb0])

            @pl.when(gp + 2 < NGP)
            def _():
                issue_group(gp + 2, fb0)

    @pl.when(base + (NGP - 2) * OG < P)
    def _():
        wait_out(0)

    @pl.when(base + (NGP - 1) * OG < P)
    def _():
        wait_out(1)


_sc_kernel = pl.kernel(
    _sc_body,
    out_type=jax.ShapeDtypeStruct((P, D), jnp.float32),
    mesh=plsc.VectorSubcoreMesh(
        core_axis_name="c", subcore_axis_name="s", num_cores=NC, num_subcores=NS),
    scratch_types=[
        pltpu.VMEM_SHARED((TROWS, D), jnp.float32),
        pltpu.VMEM((1, CP * M), jnp.int32),
        pltpu.VMEM((1, CP * M), jnp.int32),
        pltpu.VMEM((CP * M, D), jnp.float32),
        pltpu.VMEM((CP * M, D), jnp.float32),
        pltpu.VMEM((16, D), jnp.float32),
        pltpu.VMEM((16, D), jnp.float32),
        pltpu.VMEM((OG, 16), jnp.float32),
        pltpu.VMEM((OG, 16), jnp.float32),
        pltpu.VMEM((OG, D), jnp.float32),
        pltpu.VMEM((OG, D), jnp.float32),
        pltpu.SemaphoreType.DMA,
        pltpu.SemaphoreType.DMA,
        pltpu.SemaphoreType.DMA,
        pltpu.SemaphoreType.DMA,
        pltpu.SemaphoreType.DMA,
        pltpu.SemaphoreType.DMA,
        pltpu.SemaphoreType.DMA,
        pltpu.SemaphoreType.DMA,
        pltpu.SemaphoreType.DMA,
        pltpu.SemaphoreType.DMA,
    ],
)


@jax.jit
def kernel(pointfeat, adj_idx, adj_weights):
    table = jnp.zeros((TROWS, D), jnp.float32).at[1:P + 1].set(pointfeat)
    adj_p = jnp.zeros((P_PAD, M), jnp.int32).at[:P].set(adj_idx)
    adj_r = adj_p.reshape(NW * NCH, 1, CP * M)
    w_p = jnp.zeros((P_PAD,), jnp.float32).at[:P].set(adj_weights[:, 0])
    w_r = jnp.broadcast_to(w_p[:, None], (P_PAD, 16)).reshape(NW * NGP, OG, 16)
    return _sc_kernel(table, adj_r, w_r)


# speedup vs baseline: 5.5799x; 1.0385x over previous
import jax
import jax.numpy as jnp
from jax import lax
from jax.experimental import pallas as pl
from jax.experimental.pallas import tpu as pltpu
from jax.experimental.pallas import tpu_sc as plsc

P = 10000
M = 32
D = 128

NC = 2
NS = 16
NW = NC * NS
PW = 320
P_PAD = NW * PW
CP = 4
OG = 8
NGP = PW // OG
DG = D // 16
TROWS = 10112
SROWS = TROWS // NS


def _sc_body(table, adj, w, out_hbm,
             tsh, ix0, ix1, r0, r1, f0, f1, w0, w1, ob0, ob1,
             is0, is1, gs0, gs1, fs0, fs1, ws0, ws1, os0, os1):
    c = lax.axis_index("c")
    s = lax.axis_index("s")
    wid = s * NC + c
    base = wid * PW

    ix = [ix0, ix1]
    rows = [r0, r1]
    ft = [f0, f1]
    wb = [w0, w1]
    ob = [ob0, ob1]
    isx = [is0, is1]
    gsm = [gs0, gs1]
    fsm = [fs0, fs1]
    wsm = [ws0, ws1]
    osm = [os0, os1]

    def real(gp):
        return base + gp * OG < P

    def issue_idx(gp, fb):
        pltpu.async_copy(adj.at[wid * NGP + gp], ix[fb], isx[fb])

    def wait_idx(gp, fb):
        pltpu.make_async_copy(adj.at[wid * NGP + gp], ix[fb], isx[fb]).wait()

    def issue_gather(fb, half):
        pltpu.async_copy(tsh.at[ix[fb].at[half]], rows[half], gsm[half])

    def wait_gather(fb, half):
        pltpu.make_async_copy(
            tsh.at[ix[fb].at[half]], rows[half], gsm[half]).wait()

    def feat_idx(gp):
        lanes = lax.iota(jnp.int32, 16)
        return jnp.where(lanes < OG,
                         jnp.minimum(base + gp * OG + 1 + lanes, P), 0)

    def issue_group(gp, fb):
        pltpu.async_copy(tsh.at[feat_idx(gp)], ft[fb], fsm[fb])
        pltpu.async_copy(w.at[wid * NGP + gp], wb[fb], wsm[fb])

    def wait_group(gp, fb):
        pltpu.make_async_copy(tsh.at[feat_idx(gp)], ft[fb], fsm[fb]).wait()
        pltpu.make_async_copy(w.at[wid * NGP + gp], wb[fb], wsm[fb]).wait()

    def wait_out(fb):
        pltpu.make_async_copy(ob[fb], out_hbm.at[pl.ds(base, OG)], osm[fb]).wait()

    def compute(fb, half):
        @pl.loop(0, CP)
        def _(i):
            ri = i * M
            rbuf = rows[half]
            acc = [rbuf[ri, pl.ds(d * 16, 16)] for d in range(DG)]
            for m in range(1, M):
                for d in range(DG):
                    acc[d] = acc[d] + rbuf[ri + m, pl.ds(d * 16, 16)]
            pg = half * CP + i
            rw = 1.0 / (wb[fb][pg, pl.ds(0, 16)] + 1e-10)
            for d in range(DG):
                nei = acc[d] * rw
                df = nei - ft[fb][pg, pl.ds(d * 16, 16)]
                ob[fb][pg, pl.ds(d * 16, 16)] = df * df

    issue_idx(0, 0)

    pltpu.sync_copy(table.at[pl.ds(s * SROWS, SROWS)],
                    tsh.at[pl.ds(s * SROWS, SROWS)])
    plsc.subcore_barrier()

    wait_idx(0, 0)
    issue_gather(0, 0)
    issue_gather(0, 1)
    issue_group(0, 0)
    issue_group(1, 1)

    @pl.loop(0, NGP, step=2)
    def _(gp0):
        for fb in range(2):
            gp = gp0 + fb
            nxt = jnp.logical_and(gp + 1 < NGP, real(gp + 1))

            @pl.when(nxt)
            def _():
                issue_idx(gp + 1, 1 - fb)

            @pl.when(real(gp))
            def _():
                wait_gather(fb, 0)

            wait_group(gp, fb)

            @pl.when(jnp.logical_and(gp >= 2, real(gp - 2)))
            def _():
                wait_out(fb)

            compute(fb, 0)

            @pl.when(nxt)
            def _():
                wait_idx(gp + 1, 1 - fb)
                issue_gather(1 - fb, 0)

            @pl.when(real(gp))
            def _():
                wait_gather(fb, 1)

            compute(fb, 1)

            @pl.when(real(gp))
            def _():
                pltpu.async_copy(
                    ob[fb], out_hbm.at[pl.ds(base + gp * OG, OG)], osm[fb])

            @pl.when(nxt)
            def _():
                issue_gather(1 - fb, 1)

            @pl.when(gp + 2 < NGP)
            def _():
                issue_group(gp + 2, fb)

    @pl.when(real(NGP - 2))
    def _():
        wait_out(0)

    @pl.when(real(NGP - 1))
    def _():
        wait_out(1)


_sc_kernel = pl.kernel(
    _sc_body,
    out_type=jax.ShapeDtypeStruct((P, D), jnp.float32),
    mesh=plsc.VectorSubcoreMesh(
        core_axis_name="c", subcore_axis_name="s", num_cores=NC, num_subcores=NS),
    scratch_types=[
        pltpu.VMEM_SHARED((TROWS, D), jnp.float32),
        pltpu.VMEM((2, CP * M), jnp.int32),
        pltpu.VMEM((2, CP * M), jnp.int32),
        pltpu.VMEM((CP * M, D), jnp.float32),
        pltpu.VMEM((CP * M, D), jnp.float32),
        pltpu.VMEM((16, D), jnp.float32),
        pltpu.VMEM((16, D), jnp.float32),
        pltpu.VMEM((OG, 16), jnp.float32),
        pltpu.VMEM((OG, 16), jnp.float32),
        pltpu.VMEM((OG, D), jnp.float32),
        pltpu.VMEM((OG, D), jnp.float32),
        pltpu.SemaphoreType.DMA,
        pltpu.SemaphoreType.DMA,
        pltpu.SemaphoreType.DMA,
        pltpu.SemaphoreType.DMA,
        pltpu.SemaphoreType.DMA,
        pltpu.SemaphoreType.DMA,
        pltpu.SemaphoreType.DMA,
        pltpu.SemaphoreType.DMA,
        pltpu.SemaphoreType.DMA,
        pltpu.SemaphoreType.DMA,
    ],
)


@jax.jit
def kernel(pointfeat, adj_idx, adj_weights):
    table = jnp.zeros((TROWS, D), jnp.float32).at[1:P + 1].set(pointfeat)
    adj_r = adj_idx.reshape(P // OG, 2, CP * M)
    w_p = jnp.zeros((P_PAD,), jnp.float32).at[:P].set(adj_weights[:, 0])
    w_r = jnp.broadcast_to(w_p[:, None], (P_PAD, 16)).reshape(NW * NGP, OG, 16)
    return _sc_kernel(table, adj_r, w_r)


# speedup vs baseline: 5.6695x; 1.0161x over previous
import jax
import jax.numpy as jnp
from jax import lax
from jax.experimental import pallas as pl
from jax.experimental.pallas import tpu as pltpu
from jax.experimental.pallas import tpu_sc as plsc

P = 10000
M = 32
D = 128

NC = 2
NS = 16
NW = NC * NS
PW = 320
P_PAD = NW * PW
CP = 4
OG = 8
NGP = PW // OG
DG = D // 16
TROWS = 10016
ZR = 10008
SROWS = 632


def _sc_body(table, adj, w, out_hbm,
             tsh, ix0, ix1, r0, r1, f0, f1, w0, w1, ob0, ob1,
             is0, is1, gs0, gs1, fs0, fs1, ws0, ws1, os0, os1):
    c = lax.axis_index("c")
    s = lax.axis_index("s")
    wid = s * NC + c
    base = wid * PW

    ix = [ix0, ix1]
    rows = [r0, r1]
    ft = [f0, f1]
    wb = [w0, w1]
    ob = [ob0, ob1]
    isx = [is0, is1]
    gsm = [gs0, gs1]
    fsm = [fs0, fs1]
    wsm = [ws0, ws1]
    osm = [os0, os1]

    def real(gp):
        return base + gp * OG < P

    def issue_idx(gp, fb):
        pltpu.async_copy(adj.at[wid * NGP + gp], ix[fb], isx[fb])

    def wait_idx(gp, fb):
        pltpu.make_async_copy(adj.at[wid * NGP + gp], ix[fb], isx[fb]).wait()

    def issue_gather(fb, half):
        pltpu.async_copy(tsh.at[ix[fb].at[half]], rows[half], gsm[half])

    def wait_gather(fb, half):
        pltpu.make_async_copy(
            tsh.at[ix[fb].at[half]], rows[half], gsm[half]).wait()

    def feat_idx(gp):
        lanes = lax.iota(jnp.int32, 16)
        return jnp.where(lanes < OG,
                         jnp.minimum(base + gp * OG + lanes, P - 1), ZR)

    def issue_group(gp, fb):
        pltpu.async_copy(tsh.at[feat_idx(gp)], ft[fb], fsm[fb])
        pltpu.async_copy(w.at[wid * NGP + gp], wb[fb], wsm[fb])

    def wait_group(gp, fb):
        pltpu.make_async_copy(tsh.at[feat_idx(gp)], ft[fb], fsm[fb]).wait()
        pltpu.make_async_copy(w.at[wid * NGP + gp], wb[fb], wsm[fb]).wait()

    def wait_out(fb):
        pltpu.make_async_copy(ob[fb], out_hbm.at[pl.ds(base, OG)], osm[fb]).wait()

    def compute(fb, half):
        @pl.loop(0, CP)
        def _(i):
            ri = i * M
            rbuf = rows[half]
            acc = [rbuf[ri, pl.ds(d * 16, 16)] for d in range(DG)]
            for m in range(1, M):
                for d in range(DG):
                    acc[d] = acc[d] + rbuf[ri + m, pl.ds(d * 16, 16)]
            pg = half * CP + i
            rw = 1.0 / (wb[fb][pg, pl.ds(0, 16)] + 1e-10)
            for d in range(DG):
                nei = acc[d] * rw
                df = nei - ft[fb][pg, pl.ds(d * 16, 16)]
                ob[fb][pg, pl.ds(d * 16, 16)] = df * df

    def remap_idx(fb):
        for h in range(2):
            for q in range(OG):
                a = ix[fb][h, pl.ds(q * 16, 16)]
                ix[fb][h, pl.ds(q * 16, 16)] = jnp.where(a == 0, ZR, a - 1)

    issue_idx(0, 0)

    @pl.when(s < NS - 1)
    def _():
        pltpu.sync_copy(table.at[pl.ds(s * SROWS, SROWS)],
                        tsh.at[pl.ds(s * SROWS, SROWS)])

    @pl.when(s == NS - 1)
    def _():
        pltpu.sync_copy(table.at[pl.ds((NS - 1) * SROWS, P - (NS - 1) * SROWS)],
                        tsh.at[pl.ds((NS - 1) * SROWS, P - (NS - 1) * SROWS)])

    @pl.when(s == 0)
    def _():
        for r in range(OG):
            for d in range(DG):
                ob0[r, pl.ds(d * 16, 16)] = jnp.zeros((16,), jnp.float32)
        pltpu.sync_copy(ob0, tsh.at[pl.ds(ZR, OG)])

    plsc.subcore_barrier()

    wait_idx(0, 0)
    remap_idx(0)
    issue_gather(0, 0)
    issue_gather(0, 1)
    issue_group(0, 0)
    issue_group(1, 1)

    @pl.loop(0, NGP, step=2)
    def _(gp0):
        for fb in range(2):
            gp = gp0 + fb
            nxt = jnp.logical_and(gp + 1 < NGP, real(gp + 1))

            @pl.when(nxt)
            def _():
                issue_idx(gp + 1, 1 - fb)

            @pl.when(real(gp))
            def _():
                wait_gather(fb, 0)

            wait_group(gp, fb)

            @pl.when(jnp.logical_and(gp >= 2, real(gp - 2)))
            def _():
                wait_out(fb)

            compute(fb, 0)

            @pl.when(nxt)
            def _():
                wait_idx(gp + 1, 1 - fb)
                remap_idx(1 - fb)
                issue_gather(1 - fb, 0)

            @pl.when(real(gp))
            def _():
                wait_gather(fb, 1)

            compute(fb, 1)

            @pl.when(real(gp))
            def _():
                pltpu.async_copy(
                    ob[fb], out_hbm.at[pl.ds(base + gp * OG, OG)], osm[fb])

            @pl.when(nxt)
            def _():
                issue_gather(1 - fb, 1)

            @pl.when(gp + 2 < NGP)
            def _():
                issue_group(gp + 2, fb)

    @pl.when(real(NGP - 2))
    def _():
        wait_out(0)

    @pl.when(real(NGP - 1))
    def _():
        wait_out(1)


_sc_kernel = pl.kernel(
    _sc_body,
    out_type=jax.ShapeDtypeStruct((P, D), jnp.float32),
    mesh=plsc.VectorSubcoreMesh(
        core_axis_name="c", subcore_axis_name="s", num_cores=NC, num_subcores=NS),
    scratch_types=[
        pltpu.VMEM_SHARED((TROWS, D), jnp.float32),
        pltpu.VMEM((2, CP * M), jnp.int32),
        pltpu.VMEM((2, CP * M), jnp.int32),
        pltpu.VMEM((CP * M, D), jnp.float32),
        pltpu.VMEM((CP * M, D), jnp.float32),
        pltpu.VMEM((16, D), jnp.float32),
        pltpu.VMEM((16, D), jnp.float32),
        pltpu.VMEM((OG, 16), jnp.float32),
        pltpu.VMEM((OG, 16), jnp.float32),
        pltpu.VMEM((OG, D), jnp.float32),
        pltpu.VMEM((OG, D), jnp.float32),
        pltpu.SemaphoreType.DMA,
        pltpu.SemaphoreType.DMA,
        pltpu.SemaphoreType.DMA,
        pltpu.SemaphoreType.DMA,
        pltpu.SemaphoreType.DMA,
        pltpu.SemaphoreType.DMA,
        pltpu.SemaphoreType.DMA,
        pltpu.SemaphoreType.DMA,
        pltpu.SemaphoreType.DMA,
        pltpu.SemaphoreType.DMA,
    ],
)


@jax.jit
def kernel(pointfeat, adj_idx, adj_weights):
    adj_r = adj_idx.reshape(P // OG, 2, CP * M)
    w_p = jnp.zeros((P_PAD,), jnp.float32).at[:P].set(adj_weights[:, 0])
    w_r = jnp.broadcast_to(w_p[:, None], (P_PAD, 16)).reshape(NW * NGP, OG, 16)
    return _sc_kernel(pointfeat, adj_r, w_r)


# speedup vs baseline: 5.8138x; 1.0255x over previous
import jax
import jax.numpy as jnp
from jax import lax
from jax.experimental import pallas as pl
from jax.experimental.pallas import tpu as pltpu
from jax.experimental.pallas import tpu_sc as plsc

P = 10000
M = 32
D = 128

NC = 2
NS = 16
NW = NC * NS
PW = 320
P_PAD = NW * PW
CP = 4
OG = 8
NGP = PW // OG
DG = D // 16
TROWS = 10016
ZR = 10008
SROWS = 632


def _sc_body(table, adj, w, out_hbm,
             tsh, ix0, ix1, jx0, jx1, r0, r1, f0, f1, w0, w1, ob0, ob1,
             is0, is1, gs0, gs1, fs0, fs1, ws0, ws1, os0, os1):
    c = lax.axis_index("c")
    s = lax.axis_index("s")
    wid = s * NC + c
    base = wid * PW

    ix = [ix0, ix1]
    jx = [jx0, jx1]
    rows = [r0, r1]
    ft = [f0, f1]
    wb = [w0, w1]
    ob = [ob0, ob1]
    isx = [is0, is1]
    gsm = [gs0, gs1]
    fsm = [fs0, fs1]
    wsm = [ws0, ws1]
    osm = [os0, os1]

    def real(gp):
        return base + gp * OG < P

    def issue_idx(gp, fb):
        pltpu.async_copy(adj.at[pl.ds(base + gp * OG, OG)], ix[fb], isx[fb])

    def wait_idx(gp, fb):
        pltpu.make_async_copy(
            adj.at[pl.ds(base + gp * OG, OG)], ix[fb], isx[fb]).wait()

    def gref(fb, half):
        return jx[fb].at[half]

    def issue_gather(fb, half):
        pltpu.async_copy(tsh.at[gref(fb, half)], rows[half], gsm[half])

    def wait_gather(fb, half):
        pltpu.make_async_copy(
            tsh.at[gref(fb, half)], rows[half], gsm[half]).wait()

    def feat_idx(gp):
        lanes = lax.iota(jnp.int32, 16)
        return jnp.where(lanes < OG,
                         jnp.minimum(base + gp * OG + lanes, P - 1), ZR)

    def issue_group(gp, fb):
        pltpu.async_copy(tsh.at[feat_idx(gp)], ft[fb], fsm[fb])
        pltpu.async_copy(w.at[wid * NGP + gp], wb[fb], wsm[fb])

    def wait_group(gp, fb):
        pltpu.make_async_copy(tsh.at[feat_idx(gp)], ft[fb], fsm[fb]).wait()
        pltpu.make_async_copy(w.at[wid * NGP + gp], wb[fb], wsm[fb]).wait()

    def wait_out(fb):
        pltpu.make_async_copy(ob[fb], out_hbm.at[pl.ds(base, OG)], osm[fb]).wait()

    def compute(fb, half):
        @pl.loop(0, CP)
        def _(i):
            ri = i * M
            rbuf = rows[half]
            acc = [rbuf[ri, pl.ds(d * 16, 16)] for d in range(DG)]
            for m in range(1, M):
                for d in range(DG):
                    acc[d] = acc[d] + rbuf[ri + m, pl.ds(d * 16, 16)]
            pg = half * CP + i
            rw = 1.0 / (wb[fb][pg, pl.ds(0, 16)] + 1e-10)
            for d in range(DG):
                nei = acc[d] * rw
                df = nei - ft[fb][pg, pl.ds(d * 16, 16)]
                ob[fb][pg, pl.ds(d * 16, 16)] = df * df

    def remap_idx(fb):
        for h in range(2):
            for q in range(OG):
                a = ix[fb][4 * h + q // 2, pl.ds((q % 2) * 16, 16)]
                jx[fb][h, pl.ds(q * 16, 16)] = jnp.where(a == 0, ZR, a - 1)

    issue_idx(0, 0)

    @pl.when(s < NS - 1)
    def _():
        pltpu.sync_copy(table.at[pl.ds(s * SROWS, SROWS)],
                        tsh.at[pl.ds(s * SROWS, SROWS)])

    @pl.when(s == NS - 1)
    def _():
        pltpu.sync_copy(table.at[pl.ds((NS - 1) * SROWS, P - (NS - 1) * SROWS)],
                        tsh.at[pl.ds((NS - 1) * SROWS, P - (NS - 1) * SROWS)])

    @pl.when(s == 0)
    def _():
        for r in range(OG):
            for d in range(DG):
                ob0[r, pl.ds(d * 16, 16)] = jnp.zeros((16,), jnp.float32)
        pltpu.sync_copy(ob0, tsh.at[pl.ds(ZR, OG)])

    plsc.subcore_barrier()

    wait_idx(0, 0)
    remap_idx(0)
    issue_gather(0, 0)
    issue_gather(0, 1)
    issue_group(0, 0)
    issue_group(1, 1)

    @pl.loop(0, NGP, step=2)
    def _(gp0):
        for fb in range(2):
            gp = gp0 + fb
            nxt = jnp.logical_and(gp + 1 < NGP, real(gp + 1))

            @pl.when(nxt)
            def _():
                issue_idx(gp + 1, 1 - fb)

            @pl.when(real(gp))
            def _():
                wait_gather(fb, 0)

            wait_group(gp, fb)

            @pl.when(jnp.logical_and(gp >= 2, real(gp - 2)))
            def _():
                wait_out(fb)

            compute(fb, 0)

            @pl.when(nxt)
            def _():
                wait_idx(gp + 1, 1 - fb)
                remap_idx(1 - fb)
                issue_gather(1 - fb, 0)

            @pl.when(real(gp))
            def _():
                wait_gather(fb, 1)

            compute(fb, 1)

            @pl.when(real(gp))
            def _():
                pltpu.async_copy(
                    ob[fb], out_hbm.at[pl.ds(base + gp * OG, OG)], osm[fb])

            @pl.when(nxt)
            def _():
                issue_gather(1 - fb, 1)

            @pl.when(gp + 2 < NGP)
            def _():
                issue_group(gp + 2, fb)

    @pl.when(real(NGP - 2))
    def _():
        wait_out(0)

    @pl.when(real(NGP - 1))
    def _():
        wait_out(1)


_sc_kernel = pl.kernel(
    _sc_body,
    out_type=jax.ShapeDtypeStruct((P, D), jnp.float32),
    mesh=plsc.VectorSubcoreMesh(
        core_axis_name="c", subcore_axis_name="s", num_cores=NC, num_subcores=NS),
    scratch_types=[
        pltpu.VMEM_SHARED((TROWS, D), jnp.float32),
        pltpu.VMEM((OG, M), jnp.int32),
        pltpu.VMEM((OG, M), jnp.int32),
        pltpu.VMEM((2, CP * M), jnp.int32),
        pltpu.VMEM((2, CP * M), jnp.int32),
        pltpu.VMEM((CP * M, D), jnp.float32),
        pltpu.VMEM((CP * M, D), jnp.float32),
        pltpu.VMEM((16, D), jnp.float32),
        pltpu.VMEM((16, D), jnp.float32),
        pltpu.VMEM((OG, 16), jnp.float32),
        pltpu.VMEM((OG, 16), jnp.float32),
        pltpu.VMEM((OG, D), jnp.float32),
        pltpu.VMEM((OG, D), jnp.float32),
        pltpu.SemaphoreType.DMA,
        pltpu.SemaphoreType.DMA,
        pltpu.SemaphoreType.DMA,
        pltpu.SemaphoreType.DMA,
        pltpu.SemaphoreType.DMA,
        pltpu.SemaphoreType.DMA,
        pltpu.SemaphoreType.DMA,
        pltpu.SemaphoreType.DMA,
        pltpu.SemaphoreType.DMA,
        pltpu.SemaphoreType.DMA,
    ],
)


@jax.jit
def kernel(pointfeat, adj_idx, adj_weights):
    w_p = jnp.zeros((P_PAD,), jnp.float32).at[:P].set(adj_weights[:, 0])
    w_r = jnp.broadcast_to(w_p[:, None], (P_PAD, 16)).reshape(NW * NGP, OG, 16)
    return _sc_kernel(pointfeat, adj_idx, w_r)
